# baseline (speedup 1.0000x reference)
import functools

import jax
import jax.numpy as jnp
from jax import lax
from jax.experimental import pallas as pl
from jax.experimental.pallas import tpu as pltpu
from jax.experimental.pallas import tpu_sc as plsc

MAXLEN = 200
EMBED = 64
BATCH = 1024
VOCAB = 1000000

_INFO = plsc.get_sparse_core_info()
NC = _INFO.num_cores
NS = _INFO.num_subcores
NW = NC * NS
L = 16

_PARAMS = pltpu.CompilerParams(
    use_tc_tiling_on_sc=True, needs_layout_passes=False)
_MESH = dict(core_axis_name="c", subcore_axis_name="s")

NBB = BATCH // 128
NTB = NW // NBB
TROWS = MAXLEN // NTB


def _gather_body(scr, xt, posp, out_t, xv, posv, gidx, colb, posbuf,
                 gbuf, obuf, gsem, osem):
    wid = lax.axis_index("s") * NC + lax.axis_index("c")
    tb = wid // NBB
    bb = wid % NBB
    b0 = pl.multiple_of(bb * 128, 128)
    t0 = tb * TROWS
    iota = lax.iota(jnp.int32, L)

    pltpu.sync_copy(xt.at[:, pl.ds(b0, 128)], xv)
    pltpu.sync_copy(posp, posv)

    def fire_gather(i, p):
        t = t0 + i
        for g in range(8):
            sl = pl.ds(g * L, L)
            v = xv[t, sl]
            gidx[sl] = lax.shift_right_logical(v, 1)
        pltpu.async_copy(scr.at[gidx], gbuf.at[p], gsem.at[p])

    def t_body(i, carry):
        p = lax.rem(i, 2)
        t = t0 + i
        pltpu.make_async_copy(scr.at[gidx], gbuf.at[p], gsem.at[p]).wait()
        cb_local = []
        for g in range(8):
            v = xv[t, pl.ds(g * L, L)]
            cb_local.append(lax.mul(lax.bitwise_and(v, 1), EMBED))
        for dd in range(EMBED // L):
            pv = posv[t, pl.ds(dd * L, L)]
            for k in range(L):
                kvec = jnp.full((L,), k, jnp.int32)
                posbuf[dd * L + k, pl.ds(0, L)] = pv.at[kvec].get(
                    mode="promise_in_bounds")
        @pl.when(i + 1 < TROWS)
        def _():
            fire_gather(i + 1, 1 - p)
        @pl.when(i >= 2)
        def _():
            pltpu.make_async_copy(
                obuf.at[p], out_t.at[t, :, pl.ds(b0, 128)], osem.at[p]).wait()
        @plsc.parallel_loop(0, EMBED, unroll=4)
        def _(d):
            pvec = posbuf[d, pl.ds(0, L)]
            for g in range(8):
                sl = pl.ds(g * L, L)
                jv = iota + g * L
                cv = cb_local[g] + d
                vals = plsc.load_gather(gbuf.at[p], [jv, cv])
                obuf[p, d, sl] = vals + pvec
        pltpu.async_copy(obuf.at[p], out_t.at[t, :, pl.ds(b0, 128)],
                         osem.at[p])
        return carry

    fire_gather(0, 0)
    lax.fori_loop(0, TROWS, t_body, 0)
    t_last = t0 + TROWS - 1
    pltpu.make_async_copy(
        obuf.at[lax.rem(TROWS, 2)],
        out_t.at[t_last, :, pl.ds(b0, 128)],
        osem.at[lax.rem(TROWS, 2)]).wait()
    pltpu.make_async_copy(
        obuf.at[lax.rem(TROWS + 1, 2)],
        out_t.at[t_last, :, pl.ds(b0, 128)],
        osem.at[lax.rem(TROWS + 1, 2)]).wait()


@functools.partial(jax.jit, static_argnames=())
def kernel(x, token_table, pos_table):
    xt = x.T.astype(jnp.int32)
    posp = jnp.pad(pos_table, ((0, 0), (0, 64)))

    scratch = token_table.reshape(VOCAB // 2, 2 * EMBED)

    gather = pl.kernel(
        _gather_body,
        out_type=jax.ShapeDtypeStruct((MAXLEN, EMBED, BATCH), jnp.float32),
        mesh=plsc.VectorSubcoreMesh(**_MESH),
        compiler_params=_PARAMS,
        scratch_types=[
            pltpu.VMEM((MAXLEN, 128), jnp.int32),
            pltpu.VMEM((MAXLEN, 128), jnp.float32),
            pltpu.VMEM((128,), jnp.int32),
            pltpu.VMEM((128,), jnp.int32),
            pltpu.VMEM((EMBED, 128), jnp.float32),
            pltpu.VMEM((2, 128, 128), jnp.float32),
            pltpu.VMEM((2, EMBED, 128), jnp.float32),
            pltpu.SemaphoreType.DMA((2,)),
            pltpu.SemaphoreType.DMA((2,)),
        ],
    )
    out_t = gather(scratch, xt, posp)
    return out_t.transpose(2, 0, 1)
